# baseline (speedup 1.0000x reference)
import jax
import jax.numpy as jnp
from jax import lax
from jax.experimental import pallas as pl
from jax.experimental.pallas import tpu as pltpu
from jax.experimental.pallas import tpu_sc as plsc

N = 10000
E = 160000
R = 3
C = 256
NP = 10240
H = 128
NT = 16
NCORE = 2
CH = 128
NCHT = 80
EPT = NCHT * CH
EPAD = NT * EPT
NCHUNKS = EPAD // CH
BN = 1280
GRID = NP // BN
STRIPE = NP // NT

_scmesh = plsc.VectorSubcoreMesh(core_axis_name="c", subcore_axis_name="s")


def _fill(buf, val, nrows):
    v = jnp.full((16,), val, jnp.float32)

    def row(i, carry):
        for j in range(H // 16):
            buf[i, pl.ds(j * 16, 16)] = v
        return carry
    lax.fori_loop(0, nrows, row, None)


def _deg_body(dst_hbm, out_hbm, idx_v, ones_v, zero_v, acc):
    c = lax.axis_index("c")
    s = lax.axis_index("s")
    _fill(ones_v, 1.0, CH)
    _fill(zero_v, 0.0, CH)
    sl = pl.ds(s * STRIPE, STRIPE)
    w = s * NCORE + c
    for r in range(R):
        for k in range(STRIPE // CH):
            pltpu.sync_copy(zero_v, acc.at[pl.ds(s * STRIPE + k * CH, CH)])
        plsc.subcore_barrier()

        def chunk(j, carry, _r=r):
            k = j * (NT * NCORE) + w
            pltpu.sync_copy(dst_hbm.at[pl.ds(_r * EPAD + k * CH, CH)],
                            idx_v)
            pltpu.sync_copy(ones_v, acc.at[idx_v], add=True)
            return carry
        lax.fori_loop(0, NCHUNKS // (NT * NCORE), chunk, None)
        plsc.subcore_barrier()
        pltpu.sync_copy(acc.at[sl], out_hbm.at[c, r, sl])
        plsc.subcore_barrier()


_deg_kernel = pl.kernel(
    _deg_body,
    out_type=jax.ShapeDtypeStruct((NCORE, R, NP, H), jnp.float32),
    mesh=_scmesh,
    scratch_types=[
        pltpu.VMEM((CH,), jnp.int32),
        pltpu.VMEM((CH, H), jnp.float32),
        pltpu.VMEM((CH, H), jnp.float32),
        pltpu.VMEM_SHARED((NP, H), jnp.float32),
    ],
)


def _agg_body(gflat_hbm, comb_hbm, out_hbm,
              idx_v, rows0_v, rows1_v, sem0, sem1, accsp):
    c = lax.axis_index("c")
    s = lax.axis_index("s")
    sl = pl.ds(s * STRIPE, STRIPE)
    rows = (rows0_v, rows1_v)
    sems = (sem0, sem1)
    HCH = NCHT // 2
    for r in range(R):
        pltpu.sync_copy(
            gflat_hbm.at[pl.ds((2 * r + c) * NP + s * STRIPE, STRIPE)],
            accsp.at[sl])
        blk = (c * R + r) * NT + s
        first = True
        for hh in (0, 1):
            pltpu.sync_copy(comb_hbm.at[blk, pl.ds(hh * 2 * HCH, 2 * HCH)],
                            idx_v)
            pltpu.async_copy(gflat_hbm.at[idx_v.at[0]], rows0_v, sem0)
            if first:
                plsc.subcore_barrier()
                first = False

            def pair(i, carry):
                for b in (0, 1):
                    j = 2 * i + b

                    @pl.when(j < HCH - 1)
                    def _():
                        pltpu.async_copy(gflat_hbm.at[idx_v.at[2 * (j + 1)]],
                                         rows[1 - b], sems[1 - b])
                    pltpu.make_async_copy(gflat_hbm.at[pl.ds(0, CH)],
                                          rows[b], sems[b]).wait()
                    pltpu.sync_copy(rows[b], accsp.at[idx_v.at[2 * j + 1]],
                                    add=True)
                return carry
            lax.fori_loop(0, HCH // 2, pair, None)
        plsc.subcore_barrier()
        pltpu.sync_copy(accsp.at[sl], out_hbm.at[r, c, sl])
        plsc.subcore_barrier()


_agg_kernel = pl.kernel(
    _agg_body,
    out_type=jax.ShapeDtypeStruct((R, NCORE, NP, H), jnp.float32),
    mesh=_scmesh,
    scratch_types=[
        pltpu.VMEM((NCHT, CH), jnp.int32),
        pltpu.VMEM((CH, H), jnp.float32),
        pltpu.VMEM((CH, H), jnp.float32),
        pltpu.SemaphoreType.DMA,
        pltpu.SemaphoreType.DMA,
        pltpu.VMEM_SHARED((NP, H), jnp.float32),
    ],
)


def _dinv_from(degs):
    deg = 1.0 + degs[0, :, :, 0] + degs[1, :, :, 0]
    return lax.rsqrt(deg)


def _pre_body(x_ref, wg_ref, degs_ref, g_ref):
    dinv = _dinv_from(degs_ref[...])
    xb = x_ref[...]
    for r in range(R):
        h = jnp.dot(xb, wg_ref[r], preferred_element_type=jnp.float32,
                    precision=lax.Precision.HIGHEST)
        g = h * dinv[r][:, None]
        g_ref[r, 0, :, :] = g[:, :H]
        g_ref[r, 1, :, :] = g[:, H:]


_pre = pl.pallas_call(
    _pre_body,
    grid=(GRID,),
    in_specs=[
        pl.BlockSpec((BN, C), lambda i: (i, 0)),
        pl.BlockSpec((R, C, C), lambda i: (0, 0, 0)),
        pl.BlockSpec((NCORE, R, BN, H), lambda i: (0, 0, i, 0)),
    ],
    out_specs=pl.BlockSpec((R, NCORE, BN, H), lambda i: (0, 0, i, 0)),
    out_shape=jax.ShapeDtypeStruct((R, NCORE, NP, H), jnp.float32),
)


def _attention(acc, dinv, bg, w1, w2):
    hs = []
    logits = []
    for r in range(R):
        hr = jnp.concatenate([acc[r, 0], acc[r, 1]], axis=-1)
        hr = jnp.maximum(hr * dinv[r][:, None] + bg[r][None, :], 0.0)
        t = jnp.tanh(jnp.dot(hr, w1[r], preferred_element_type=jnp.float32,
                             precision=lax.Precision.HIGHEST))
        logits.append(jnp.sum(t * w2[r][None, :], axis=-1))
        hs.append(hr)
    m = jnp.maximum(jnp.maximum(logits[0], logits[1]), logits[2])
    es = [jnp.exp(a - m) for a in logits]
    ssum = es[0] + es[1] + es[2]
    y = hs[0] * (es[0] / ssum)[:, None]
    y += hs[1] * (es[1] / ssum)[:, None]
    y += hs[2] * (es[2] / ssum)[:, None]
    return y


def _mid_body(acc_ref, degs_ref, bg_ref, w1_ref, w2_ref, wgn_ref, g_ref):
    dinv = _dinv_from(degs_ref[...])
    y = _attention(acc_ref[...], dinv, bg_ref[...], w1_ref[...], w2_ref[...])
    for r in range(R):
        g2 = jnp.dot(y, wgn_ref[r], preferred_element_type=jnp.float32,
                     precision=lax.Precision.HIGHEST)
        g2 = g2 * dinv[r][:, None]
        g_ref[r, 0, :, :] = g2[:, :H]
        g_ref[r, 1, :, :] = g2[:, H:]


_mid = pl.pallas_call(
    _mid_body,
    grid=(GRID,),
    in_specs=[
        pl.BlockSpec((R, NCORE, BN, H), lambda i: (0, 0, i, 0)),
        pl.BlockSpec((NCORE, R, BN, H), lambda i: (0, 0, i, 0)),
        pl.BlockSpec((R, C), lambda i: (0, 0)),
        pl.BlockSpec((R, C, 64), lambda i: (0, 0, 0)),
        pl.BlockSpec((R, 64), lambda i: (0, 0)),
        pl.BlockSpec((R, C, C), lambda i: (0, 0, 0)),
    ],
    out_specs=pl.BlockSpec((R, NCORE, BN, H), lambda i: (0, 0, i, 0)),
    out_shape=jax.ShapeDtypeStruct((R, NCORE, NP, H), jnp.float32),
)


def _post_body(acc_ref, degs_ref, bg_ref, w1_ref, w2_ref, pw_ref, pb_ref,
               o_ref):
    dinv = _dinv_from(degs_ref[...])
    y = _attention(acc_ref[...], dinv, bg_ref[...], w1_ref[...], w2_ref[...])
    o_ref[...] = (jnp.sum(y * pw_ref[...][None, :], axis=-1)
                  + pb_ref[0])[:, None]


_post = pl.pallas_call(
    _post_body,
    grid=(GRID,),
    in_specs=[
        pl.BlockSpec((R, NCORE, BN, H), lambda i: (0, 0, i, 0)),
        pl.BlockSpec((NCORE, R, BN, H), lambda i: (0, 0, i, 0)),
        pl.BlockSpec((R, C), lambda i: (0, 0)),
        pl.BlockSpec((R, C, 64), lambda i: (0, 0, 0)),
        pl.BlockSpec((R, 64), lambda i: (0, 0)),
        pl.BlockSpec((C,), lambda i: (0,)),
        pl.BlockSpec((1,), lambda i: (0,)),
    ],
    out_specs=pl.BlockSpec((BN, 1), lambda i: (i, 0)),
    out_shape=jax.ShapeDtypeStruct((NP, 1), jnp.float32),
)


def kernel(x, edge_index_list, Wg0, bg0, Wg1, bg1, W1_0, W2_0, W1_1, W2_1,
           pred_W, pred_b):
    ei = edge_index_list.astype(jnp.int32)
    src = ei[:, 0, :]
    dst = ei[:, 1, :]
    npad = EPAD - E
    srcp = jnp.pad(src, ((0, 0), (0, npad)))
    dstp = jnp.pad(dst, ((0, 0), (0, npad)), constant_values=NP - 1)
    offs = (jnp.arange(R, dtype=jnp.int32)[None, :, None] * 2
            + jnp.arange(NCORE, dtype=jnp.int32)[:, None, None]) * NP
    srcg = offs + srcp[None]

    xpad = jnp.pad(x, ((0, NP - N), (0, 0)))
    dstp_flat = dstp.reshape(R * EPAD)
    sg = srcg.reshape(NCORE, R, NT, NCHT, CH)
    dp = jnp.broadcast_to(dstp.reshape(1, R, NT, NCHT, CH), sg.shape)
    comb = jnp.stack([sg, dp], axis=4).reshape(NCORE * R * NT, 2 * NCHT, CH)

    degs = _deg_kernel(dstp_flat)
    g0 = _pre(xpad, Wg0, degs)
    acc0 = _agg_kernel(g0.reshape(R * NCORE * NP, H), comb)
    g1 = _mid(acc0, degs, bg0, W1_0, W2_0[:, :, 0], Wg1)
    acc1 = _agg_kernel(g1.reshape(R * NCORE * NP, H), comb)
    out = _post(acc1, degs, bg1, W1_1, W2_1[:, :, 0], pred_W[:, 0], pred_b)
    return out[:N, 0]


# speedup vs baseline: 1.0316x; 1.0316x over previous
import jax
import jax.numpy as jnp
from jax import lax
from jax.experimental import pallas as pl
from jax.experimental.pallas import tpu as pltpu
from jax.experimental.pallas import tpu_sc as plsc

N = 10000
E = 160000
R = 3
C = 256
NP = 10240
H = 128
NT = 16
NCORE = 2
CH = 128
NCHT = 80
EPT = NCHT * CH
EPAD = NT * EPT
NCHUNKS = EPAD // CH
BN = 1280
GRID = NP // BN
STRIPE = NP // NT

_scmesh = plsc.VectorSubcoreMesh(core_axis_name="c", subcore_axis_name="s")


def _fill(buf, val, nrows):
    v = jnp.full((16,), val, jnp.float32)

    def row(i, carry):
        for j in range(H // 16):
            buf[i, pl.ds(j * 16, 16)] = v
        return carry
    lax.fori_loop(0, nrows, row, None)


def _deg_body(dst_hbm, out_hbm, idx_v, ones_v, zero_v, acc):
    c = lax.axis_index("c")
    s = lax.axis_index("s")
    _fill(ones_v, 1.0, CH)
    _fill(zero_v, 0.0, CH)
    sl = pl.ds(s * STRIPE, STRIPE)
    w = s * NCORE + c
    for r in range(R):
        for k in range(STRIPE // CH):
            pltpu.sync_copy(zero_v, acc.at[pl.ds(s * STRIPE + k * CH, CH)])
        plsc.subcore_barrier()

        def chunk(j, carry, _r=r):
            k = j * (NT * NCORE) + w
            pltpu.sync_copy(dst_hbm.at[pl.ds(_r * EPAD + k * CH, CH)],
                            idx_v)
            pltpu.sync_copy(ones_v, acc.at[idx_v], add=True)
            return carry
        lax.fori_loop(0, NCHUNKS // (NT * NCORE), chunk, None)
        plsc.subcore_barrier()
        pltpu.sync_copy(acc.at[sl], out_hbm.at[c, r, sl])
        plsc.subcore_barrier()


_deg_kernel = pl.kernel(
    _deg_body,
    out_type=jax.ShapeDtypeStruct((NCORE, R, NP, H), jnp.float32),
    mesh=_scmesh,
    scratch_types=[
        pltpu.VMEM((CH,), jnp.int32),
        pltpu.VMEM((CH, H), jnp.float32),
        pltpu.VMEM((CH, H), jnp.float32),
        pltpu.VMEM_SHARED((NP, H), jnp.float32),
    ],
)


def _agg_body(gflat_hbm, comb_hbm, out_hbm,
              idx_v, rows0_v, rows1_v, sem0, sem1, accsp):
    c = lax.axis_index("c")
    s = lax.axis_index("s")
    sl = pl.ds(s * STRIPE, STRIPE)
    rows = (rows0_v, rows1_v)
    sems = (sem0, sem1)
    HCH = NCHT // 2
    for r in range(R):
        pltpu.sync_copy(
            gflat_hbm.at[pl.ds((2 * r + c) * NP + s * STRIPE, STRIPE)],
            accsp.at[sl])
        blk = (c * R + r) * NT + s
        first = True
        for hh in (0, 1):
            pltpu.sync_copy(comb_hbm.at[blk, pl.ds(hh * 2 * HCH, 2 * HCH)],
                            idx_v)
            pltpu.async_copy(gflat_hbm.at[idx_v.at[0]], rows0_v, sem0)
            if first:
                plsc.subcore_barrier()
                first = False

            def pair(i, carry):
                for b in (0, 1):
                    j = 2 * i + b

                    @pl.when(j < HCH - 1)
                    def _():
                        pltpu.async_copy(gflat_hbm.at[idx_v.at[2 * (j + 1)]],
                                         rows[1 - b], sems[1 - b])
                    pltpu.make_async_copy(gflat_hbm.at[pl.ds(0, CH)],
                                          rows[b], sems[b]).wait()
                    pltpu.sync_copy(rows[b], accsp.at[idx_v.at[2 * j + 1]],
                                    add=True)
                return carry
            lax.fori_loop(0, HCH // 2, pair, None)
        plsc.subcore_barrier()
        pltpu.sync_copy(accsp.at[sl], out_hbm.at[r, c, sl])
        plsc.subcore_barrier()


_agg_kernel = pl.kernel(
    _agg_body,
    out_type=jax.ShapeDtypeStruct((R, NCORE, NP, H), jnp.float32),
    mesh=_scmesh,
    scratch_types=[
        pltpu.VMEM((NCHT, CH), jnp.int32),
        pltpu.VMEM((CH, H), jnp.float32),
        pltpu.VMEM((CH, H), jnp.float32),
        pltpu.SemaphoreType.DMA,
        pltpu.SemaphoreType.DMA,
        pltpu.VMEM_SHARED((NP, H), jnp.float32),
    ],
)


def _dinv_from(degs):
    deg = 1.0 + degs[0, :, :, 0] + degs[1, :, :, 0]
    return lax.rsqrt(deg)


def _pre_body(x_ref, wg_ref, degs_ref, g_ref):
    dinv = _dinv_from(degs_ref[...])
    xb = x_ref[...]
    for r in range(R):
        h = jnp.dot(xb, wg_ref[r], preferred_element_type=jnp.float32)
        g = h * dinv[r][:, None]
        g_ref[r, 0, :, :] = g[:, :H]
        g_ref[r, 1, :, :] = g[:, H:]


_pre = pl.pallas_call(
    _pre_body,
    grid=(GRID,),
    in_specs=[
        pl.BlockSpec((BN, C), lambda i: (i, 0)),
        pl.BlockSpec((R, C, C), lambda i: (0, 0, 0)),
        pl.BlockSpec((NCORE, R, BN, H), lambda i: (0, 0, i, 0)),
    ],
    out_specs=pl.BlockSpec((R, NCORE, BN, H), lambda i: (0, 0, i, 0)),
    out_shape=jax.ShapeDtypeStruct((R, NCORE, NP, H), jnp.float32),
)


def _attention(acc, dinv, bg, w1, w2):
    hs = []
    logits = []
    for r in range(R):
        hr = jnp.concatenate([acc[r, 0], acc[r, 1]], axis=-1)
        hr = jnp.maximum(hr * dinv[r][:, None] + bg[r][None, :], 0.0)
        t = jnp.tanh(jnp.dot(hr, w1[r], preferred_element_type=jnp.float32))
        logits.append(jnp.sum(t * w2[r][None, :], axis=-1))
        hs.append(hr)
    m = jnp.maximum(jnp.maximum(logits[0], logits[1]), logits[2])
    es = [jnp.exp(a - m) for a in logits]
    ssum = es[0] + es[1] + es[2]
    y = hs[0] * (es[0] / ssum)[:, None]
    y += hs[1] * (es[1] / ssum)[:, None]
    y += hs[2] * (es[2] / ssum)[:, None]
    return y


def _mid_body(acc_ref, degs_ref, bg_ref, w1_ref, w2_ref, wgn_ref, g_ref):
    dinv = _dinv_from(degs_ref[...])
    y = _attention(acc_ref[...], dinv, bg_ref[...], w1_ref[...], w2_ref[...])
    for r in range(R):
        g2 = jnp.dot(y, wgn_ref[r], preferred_element_type=jnp.float32)
        g2 = g2 * dinv[r][:, None]
        g_ref[r, 0, :, :] = g2[:, :H]
        g_ref[r, 1, :, :] = g2[:, H:]


_mid = pl.pallas_call(
    _mid_body,
    grid=(GRID,),
    in_specs=[
        pl.BlockSpec((R, NCORE, BN, H), lambda i: (0, 0, i, 0)),
        pl.BlockSpec((NCORE, R, BN, H), lambda i: (0, 0, i, 0)),
        pl.BlockSpec((R, C), lambda i: (0, 0)),
        pl.BlockSpec((R, C, 64), lambda i: (0, 0, 0)),
        pl.BlockSpec((R, 64), lambda i: (0, 0)),
        pl.BlockSpec((R, C, C), lambda i: (0, 0, 0)),
    ],
    out_specs=pl.BlockSpec((R, NCORE, BN, H), lambda i: (0, 0, i, 0)),
    out_shape=jax.ShapeDtypeStruct((R, NCORE, NP, H), jnp.float32),
)


def _post_body(acc_ref, degs_ref, bg_ref, w1_ref, w2_ref, pw_ref, pb_ref,
               o_ref):
    dinv = _dinv_from(degs_ref[...])
    y = _attention(acc_ref[...], dinv, bg_ref[...], w1_ref[...], w2_ref[...])
    o_ref[...] = (jnp.sum(y * pw_ref[...][None, :], axis=-1)
                  + pb_ref[0])[:, None]


_post = pl.pallas_call(
    _post_body,
    grid=(GRID,),
    in_specs=[
        pl.BlockSpec((R, NCORE, BN, H), lambda i: (0, 0, i, 0)),
        pl.BlockSpec((NCORE, R, BN, H), lambda i: (0, 0, i, 0)),
        pl.BlockSpec((R, C), lambda i: (0, 0)),
        pl.BlockSpec((R, C, 64), lambda i: (0, 0, 0)),
        pl.BlockSpec((R, 64), lambda i: (0, 0)),
        pl.BlockSpec((C,), lambda i: (0,)),
        pl.BlockSpec((1,), lambda i: (0,)),
    ],
    out_specs=pl.BlockSpec((BN, 1), lambda i: (i, 0)),
    out_shape=jax.ShapeDtypeStruct((NP, 1), jnp.float32),
)


def kernel(x, edge_index_list, Wg0, bg0, Wg1, bg1, W1_0, W2_0, W1_1, W2_1,
           pred_W, pred_b):
    ei = edge_index_list.astype(jnp.int32)
    src = ei[:, 0, :]
    dst = ei[:, 1, :]
    npad = EPAD - E
    srcp = jnp.pad(src, ((0, 0), (0, npad)))
    dstp = jnp.pad(dst, ((0, 0), (0, npad)), constant_values=NP - 1)
    offs = (jnp.arange(R, dtype=jnp.int32)[None, :, None] * 2
            + jnp.arange(NCORE, dtype=jnp.int32)[:, None, None]) * NP
    srcg = offs + srcp[None]

    xpad = jnp.pad(x, ((0, NP - N), (0, 0)))
    dstp_flat = dstp.reshape(R * EPAD)
    sg = srcg.reshape(NCORE, R, NT, NCHT, CH)
    dp = jnp.broadcast_to(dstp.reshape(1, R, NT, NCHT, CH), sg.shape)
    comb = jnp.stack([sg, dp], axis=4).reshape(NCORE * R * NT, 2 * NCHT, CH)

    degs = _deg_kernel(dstp_flat)
    g0 = _pre(xpad, Wg0, degs)
    acc0 = _agg_kernel(g0.reshape(R * NCORE * NP, H), comb)
    g1 = _mid(acc0, degs, bg0, W1_0, W2_0[:, :, 0], Wg1)
    acc1 = _agg_kernel(g1.reshape(R * NCORE * NP, H), comb)
    out = _post(acc1, degs, bg1, W1_1, W2_1[:, :, 0], pred_W[:, 0], pred_b)
    return out[:N, 0]
